# Initial kernel scaffold; baseline (speedup 1.0000x reference)
import functools

import jax
import jax.numpy as jnp
from jax import lax
from jax.experimental import pallas as pl
from jax.experimental.pallas import tpu as pltpu
from jax.experimental.pallas import tpu_sc as plsc

_N = 10000
_NPAD = 10240
_E = 320000
_EPAD = 327680
_D = 128
_NC, _NS = 2, 16
_CHUNK = 128
_TILE_EDGES = _EPAD // (_NC * _NS)
_NCHUNK = _TILE_EDGES // _CHUNK
_ROWS_PT = _NPAD // _NS

_mesh = plsc.VectorSubcoreMesh(
    core_axis_name="c", subcore_axis_name="s", num_cores=_NC, num_subcores=_NS
)


@functools.partial(
    pl.kernel,
    out_type=jax.ShapeDtypeStruct((_NC, _NPAD, 16), jnp.float32),
    mesh=_mesh,
    scratch_types=[
        pltpu.VMEM((_NCHUNK, _CHUNK), jnp.int32),
        pltpu.VMEM((_CHUNK, 16), jnp.float32),
        pltpu.VMEM_SHARED((_NPAD, 16), jnp.float32),
    ],
)
def _deg_kernel(dst_hbm, ones_hbm, zeros_hbm, deg_hbm, dst_v, ones_v, deg_sh):
    c = lax.axis_index("c")
    s = lax.axis_index("s")
    tid = c * _NS + s
    pltpu.sync_copy(dst_hbm.at[tid], dst_v)
    pltpu.sync_copy(ones_hbm, ones_v)
    pltpu.sync_copy(
        zeros_hbm.at[pl.ds(s * _ROWS_PT, _ROWS_PT)],
        deg_sh.at[pl.ds(s * _ROWS_PT, _ROWS_PT)],
    )
    plsc.subcore_barrier()

    def body(j, carry):
        pltpu.sync_copy(ones_v, deg_sh.at[dst_v.at[j]], add=True)
        return carry

    lax.fori_loop(0, _NCHUNK, body, 0)
    plsc.subcore_barrier()
    pltpu.sync_copy(
        deg_sh.at[pl.ds(s * _ROWS_PT, _ROWS_PT)],
        deg_hbm.at[c, pl.ds(s * _ROWS_PT, _ROWS_PT)],
    )


@functools.partial(
    pl.kernel,
    out_type=jax.ShapeDtypeStruct((_NC, _NPAD, _D), jnp.float32),
    mesh=_mesh,
    scratch_types=[
        pltpu.VMEM((_NCHUNK, _CHUNK), jnp.int32),
        pltpu.VMEM((_NCHUNK, _CHUNK), jnp.int32),
        pltpu.VMEM((_CHUNK, _D), jnp.float32),
        pltpu.VMEM((_CHUNK, _D), jnp.float32),
        pltpu.VMEM_SHARED((_NPAD, _D), jnp.float32),
        pltpu.SemaphoreType.DMA,
        pltpu.SemaphoreType.DMA,
    ],
)
def _spmm_kernel(src_hbm, dst_hbm, g_hbm, zeros_hbm, out_hbm,
                 src_v, dst_v, rows_a, rows_b, y_sh, sem_a, sem_b):
    c = lax.axis_index("c")
    s = lax.axis_index("s")
    tid = c * _NS + s
    pltpu.sync_copy(src_hbm.at[tid], src_v)
    pltpu.sync_copy(dst_hbm.at[tid], dst_v)
    pltpu.sync_copy(
        zeros_hbm.at[pl.ds(s * _ROWS_PT, _ROWS_PT)],
        y_sh.at[pl.ds(s * _ROWS_PT, _ROWS_PT)],
    )
    plsc.subcore_barrier()

    def body(i, carry):
        j0 = 2 * i
        j1 = 2 * i + 1
        ha = pltpu.async_copy(g_hbm.at[src_v.at[j0]], rows_a, sem_a)
        hb = pltpu.async_copy(g_hbm.at[src_v.at[j1]], rows_b, sem_b)
        ha.wait()
        pltpu.sync_copy(rows_a, y_sh.at[dst_v.at[j0]], add=True)
        hb.wait()
        pltpu.sync_copy(rows_b, y_sh.at[dst_v.at[j1]], add=True)
        return carry

    lax.fori_loop(0, _NCHUNK // 2, body, 0)
    plsc.subcore_barrier()
    pltpu.sync_copy(
        y_sh.at[pl.ds(s * _ROWS_PT, _ROWS_PT)],
        out_hbm.at[c, pl.ds(s * _ROWS_PT, _ROWS_PT)],
    )


_BLK = 256


def _dis_of(deg_ref):
    deg = deg_ref[0, :, 0:1] + deg_ref[1, :, 0:1] + 1.0
    return lax.rsqrt(deg)


def _tc_a_body(x_ref, w_ref, deg_ref, o_ref):
    dis = _dis_of(deg_ref)
    o_ref[...] = jnp.dot(
        x_ref[...], w_ref[...], preferred_element_type=jnp.float32
    ) * dis


_tc_a = pl.pallas_call(
    _tc_a_body,
    grid=(_NPAD // _BLK,),
    in_specs=[
        pl.BlockSpec((_BLK, _D), lambda i: (i, 0)),
        pl.BlockSpec((_D, _D), lambda i: (0, 0)),
        pl.BlockSpec((_NC, _BLK, 16), lambda i: (0, i, 0)),
    ],
    out_specs=pl.BlockSpec((_BLK, _D), lambda i: (i, 0)),
    out_shape=jax.ShapeDtypeStruct((_NPAD, _D), jnp.float32),
)


def _tc_b_body(y_ref, g_ref, deg_ref, b_ref, w_ref, o_ref):
    dis = _dis_of(deg_ref)
    t = (y_ref[0] + y_ref[1] + g_ref[...]) * dis + b_ref[...]
    h = jnp.maximum(t, 0.0)
    o_ref[...] = jnp.dot(h, w_ref[...], preferred_element_type=jnp.float32) * dis


_tc_b = pl.pallas_call(
    _tc_b_body,
    grid=(_NPAD // _BLK,),
    in_specs=[
        pl.BlockSpec((_NC, _BLK, _D), lambda i: (0, i, 0)),
        pl.BlockSpec((_BLK, _D), lambda i: (i, 0)),
        pl.BlockSpec((_NC, _BLK, 16), lambda i: (0, i, 0)),
        pl.BlockSpec((1, _D), lambda i: (0, 0)),
        pl.BlockSpec((_D, _D), lambda i: (0, 0)),
    ],
    out_specs=pl.BlockSpec((_BLK, _D), lambda i: (i, 0)),
    out_shape=jax.ShapeDtypeStruct((_NPAD, _D), jnp.float32),
)


def _tc_c_body(y_ref, g_ref, deg_ref, b_ref, w_ref, bl_ref, o_ref):
    dis = _dis_of(deg_ref)
    t = (y_ref[0] + y_ref[1] + g_ref[...]) * dis + b_ref[...]
    h = jnp.maximum(t, 0.0)
    o_ref[...] = jnp.dot(
        h, w_ref[...], preferred_element_type=jnp.float32
    ) + bl_ref[...]


_tc_c = pl.pallas_call(
    _tc_c_body,
    grid=(_NPAD // _BLK,),
    in_specs=[
        pl.BlockSpec((_NC, _BLK, _D), lambda i: (0, i, 0)),
        pl.BlockSpec((_BLK, _D), lambda i: (i, 0)),
        pl.BlockSpec((_NC, _BLK, 16), lambda i: (0, i, 0)),
        pl.BlockSpec((1, _D), lambda i: (0, 0)),
        pl.BlockSpec((_D, _D), lambda i: (0, 0)),
        pl.BlockSpec((1, _D), lambda i: (0, 0)),
    ],
    out_specs=pl.BlockSpec((_BLK, _D), lambda i: (i, 0)),
    out_shape=jax.ShapeDtypeStruct((_NPAD, _D), jnp.float32),
)


@jax.jit
def _run(x, edge_index, W1, b1, W2, b2, Wl, bl):
    src = edge_index[0].astype(jnp.int32)
    dst = edge_index[1].astype(jnp.int32)
    pad = jnp.full((_EPAD - _E,), _N, jnp.int32)
    src3 = jnp.concatenate([src, pad]).reshape(_NC * _NS, _NCHUNK, _CHUNK)
    dst3 = jnp.concatenate([dst, pad]).reshape(_NC * _NS, _NCHUNK, _CHUNK)
    x_pad = jnp.concatenate([x, jnp.zeros((_NPAD - _N, _D), x.dtype)])
    ones16 = jnp.ones((_CHUNK, 16), jnp.float32)
    zeros16 = jnp.zeros((_NPAD, 16), jnp.float32)
    zeros_d = jnp.zeros((_NPAD, _D), jnp.float32)

    deg = _deg_kernel(dst3, ones16, zeros16)
    g1 = _tc_a(x_pad, W1, deg)
    y1 = _spmm_kernel(src3, dst3, g1, zeros_d)
    g2 = _tc_b(y1, g1, deg, b1.reshape(1, _D), W2)
    y2 = _spmm_kernel(src3, dst3, g2, zeros_d)
    out = _tc_c(y2, g2, deg, b2.reshape(1, _D), Wl, bl.reshape(1, _D))
    return out[:_N]


def kernel(x, edge_index, W1, b1, W2, b2, Wl, bl):
    return _run(x, edge_index, W1, b1, W2, b2, Wl, bl)


# speedup vs baseline: 8.2329x; 8.2329x over previous
import functools

import jax
import jax.numpy as jnp
from jax import lax
from jax.experimental import pallas as pl
from jax.experimental.pallas import tpu as pltpu
from jax.experimental.pallas import tpu_sc as plsc

_N = 10000
_NPAD = 10240
_E = 320000
_EPAD = 327680
_D = 128
_NC, _NS = 2, 16
_CHUNK = 128
_TILE_EDGES = _EPAD // (_NC * _NS)
_NCHUNK = _TILE_EDGES // _CHUNK
_ROWS_PT = _NPAD // _NS

_mesh = plsc.VectorSubcoreMesh(
    core_axis_name="c", subcore_axis_name="s", num_cores=_NC, num_subcores=_NS
)


@functools.partial(
    pl.kernel,
    out_type=jax.ShapeDtypeStruct((_NC, _NPAD, 16), jnp.float32),
    mesh=_mesh,
    scratch_types=[
        pltpu.VMEM((_NCHUNK, _CHUNK), jnp.int32),
        pltpu.VMEM((_CHUNK, 16), jnp.float32),
        pltpu.VMEM_SHARED((_NPAD, 16), jnp.float32),
    ],
)
def _deg_kernel(dst_hbm, ones_hbm, zeros_hbm, deg_hbm, dst_v, ones_v, deg_sh):
    c = lax.axis_index("c")
    s = lax.axis_index("s")
    tid = c * _NS + s
    pltpu.sync_copy(dst_hbm.at[tid], dst_v)
    pltpu.sync_copy(ones_hbm, ones_v)
    pltpu.sync_copy(
        zeros_hbm.at[pl.ds(s * _ROWS_PT, _ROWS_PT)],
        deg_sh.at[pl.ds(s * _ROWS_PT, _ROWS_PT)],
    )
    plsc.subcore_barrier()

    def body(j, carry):
        pltpu.sync_copy(ones_v, deg_sh.at[dst_v.at[j]], add=True)
        return carry

    lax.fori_loop(0, _NCHUNK, body, 0)
    plsc.subcore_barrier()
    pltpu.sync_copy(
        deg_sh.at[pl.ds(s * _ROWS_PT, _ROWS_PT)],
        deg_hbm.at[c, pl.ds(s * _ROWS_PT, _ROWS_PT)],
    )


_STAGES = 2
_SCHUNK = _NCHUNK // _STAGES


@functools.partial(
    pl.kernel,
    out_type=jax.ShapeDtypeStruct((_NC, _NPAD, _D), jnp.float32),
    mesh=_mesh,
    scratch_types=[
        pltpu.VMEM((_SCHUNK, _CHUNK), jnp.int32),
        pltpu.VMEM((_SCHUNK, _CHUNK), jnp.int32),
        pltpu.VMEM((_CHUNK, _D), jnp.float32),
        pltpu.VMEM((_CHUNK, _D), jnp.float32),
        pltpu.VMEM_SHARED((_NPAD, _D), jnp.float32),
        pltpu.SemaphoreType.DMA,
        pltpu.SemaphoreType.DMA,
    ],
)
def _spmm_kernel(src_hbm, dst_hbm, g_hbm, zeros_hbm, out_hbm,
                 src_v, dst_v, rows_a, rows_b, y_sh, sem_a, sem_b):
    c = lax.axis_index("c")
    s = lax.axis_index("s")
    tid = c * _NS + s
    pltpu.sync_copy(
        zeros_hbm.at[pl.ds(s * _ROWS_PT, _ROWS_PT)],
        y_sh.at[pl.ds(s * _ROWS_PT, _ROWS_PT)],
    )
    plsc.subcore_barrier()

    def body(i, carry):
        j0 = 2 * i
        j1 = 2 * i + 1
        ha = pltpu.async_copy(g_hbm.at[src_v.at[j0]], rows_a, sem_a)
        hb = pltpu.async_copy(g_hbm.at[src_v.at[j1]], rows_b, sem_b)
        ha.wait()
        pltpu.sync_copy(rows_a, y_sh.at[dst_v.at[j0]], add=True)
        hb.wait()
        pltpu.sync_copy(rows_b, y_sh.at[dst_v.at[j1]], add=True)
        return carry

    for stage in range(_STAGES):
        pltpu.sync_copy(src_hbm.at[tid, pl.ds(stage * _SCHUNK, _SCHUNK)], src_v)
        pltpu.sync_copy(dst_hbm.at[tid, pl.ds(stage * _SCHUNK, _SCHUNK)], dst_v)
        lax.fori_loop(0, _SCHUNK // 2, body, 0)
    plsc.subcore_barrier()
    pltpu.sync_copy(
        y_sh.at[pl.ds(s * _ROWS_PT, _ROWS_PT)],
        out_hbm.at[c, pl.ds(s * _ROWS_PT, _ROWS_PT)],
    )


_BLK = 256


def _dis_of(deg_ref):
    deg = deg_ref[0, :, 0:1] + deg_ref[1, :, 0:1] + 1.0
    return lax.rsqrt(deg)


def _tc_a_body(x_ref, w_ref, deg_ref, o_ref):
    dis = _dis_of(deg_ref)
    o_ref[...] = jnp.dot(
        x_ref[...], w_ref[...], preferred_element_type=jnp.float32
    ) * dis


_tc_a = pl.pallas_call(
    _tc_a_body,
    grid=(_NPAD // _BLK,),
    in_specs=[
        pl.BlockSpec((_BLK, _D), lambda i: (i, 0)),
        pl.BlockSpec((_D, _D), lambda i: (0, 0)),
        pl.BlockSpec((_NC, _BLK, 16), lambda i: (0, i, 0)),
    ],
    out_specs=pl.BlockSpec((_BLK, _D), lambda i: (i, 0)),
    out_shape=jax.ShapeDtypeStruct((_NPAD, _D), jnp.float32),
)


def _tc_b_body(y_ref, g_ref, deg_ref, b_ref, w_ref, o_ref):
    dis = _dis_of(deg_ref)
    t = (y_ref[0] + y_ref[1] + g_ref[...]) * dis + b_ref[...]
    h = jnp.maximum(t, 0.0)
    o_ref[...] = jnp.dot(h, w_ref[...], preferred_element_type=jnp.float32) * dis


_tc_b = pl.pallas_call(
    _tc_b_body,
    grid=(_NPAD // _BLK,),
    in_specs=[
        pl.BlockSpec((_NC, _BLK, _D), lambda i: (0, i, 0)),
        pl.BlockSpec((_BLK, _D), lambda i: (i, 0)),
        pl.BlockSpec((_NC, _BLK, 16), lambda i: (0, i, 0)),
        pl.BlockSpec((1, _D), lambda i: (0, 0)),
        pl.BlockSpec((_D, _D), lambda i: (0, 0)),
    ],
    out_specs=pl.BlockSpec((_BLK, _D), lambda i: (i, 0)),
    out_shape=jax.ShapeDtypeStruct((_NPAD, _D), jnp.float32),
)


def _tc_c_body(y_ref, g_ref, deg_ref, b_ref, w_ref, bl_ref, o_ref):
    dis = _dis_of(deg_ref)
    t = (y_ref[0] + y_ref[1] + g_ref[...]) * dis + b_ref[...]
    h = jnp.maximum(t, 0.0)
    o_ref[...] = jnp.dot(
        h, w_ref[...], preferred_element_type=jnp.float32
    ) + bl_ref[...]


_tc_c = pl.pallas_call(
    _tc_c_body,
    grid=(_NPAD // _BLK,),
    in_specs=[
        pl.BlockSpec((_NC, _BLK, _D), lambda i: (0, i, 0)),
        pl.BlockSpec((_BLK, _D), lambda i: (i, 0)),
        pl.BlockSpec((_NC, _BLK, 16), lambda i: (0, i, 0)),
        pl.BlockSpec((1, _D), lambda i: (0, 0)),
        pl.BlockSpec((_D, _D), lambda i: (0, 0)),
        pl.BlockSpec((1, _D), lambda i: (0, 0)),
    ],
    out_specs=pl.BlockSpec((_BLK, _D), lambda i: (i, 0)),
    out_shape=jax.ShapeDtypeStruct((_NPAD, _D), jnp.float32),
)


@jax.jit
def _run(x, edge_index, W1, b1, W2, b2, Wl, bl):
    src = edge_index[0].astype(jnp.int32)
    dst = edge_index[1].astype(jnp.int32)
    pad = jnp.full((_EPAD - _E,), _N, jnp.int32)
    src3 = jnp.concatenate([src, pad]).reshape(_NC * _NS, _NCHUNK, _CHUNK)
    dst3 = jnp.concatenate([dst, pad]).reshape(_NC * _NS, _NCHUNK, _CHUNK)
    x_pad = jnp.concatenate([x, jnp.zeros((_NPAD - _N, _D), x.dtype)])
    ones16 = jnp.ones((_CHUNK, 16), jnp.float32)
    zeros16 = jnp.zeros((_NPAD, 16), jnp.float32)
    zeros_d = jnp.zeros((_NPAD, _D), jnp.float32)

    deg = _deg_kernel(dst3, ones16, zeros16)
    g1 = _tc_a(x_pad, W1, deg)
    y1 = _spmm_kernel(src3, dst3, g1, zeros_d)
    g2 = _tc_b(y1, g1, deg, b1.reshape(1, _D), W2)
    y2 = _spmm_kernel(src3, dst3, g2, zeros_d)
    out = _tc_c(y2, g2, deg, b2.reshape(1, _D), Wl, bl.reshape(1, _D))
    return out[:_N]


def kernel(x, edge_index, W1, b1, W2, b2, Wl, bl):
    return _run(x, edge_index, W1, b1, W2, b2, Wl, bl)


# speedup vs baseline: 8.2916x; 1.0071x over previous
import functools

import jax
import jax.numpy as jnp
from jax import lax
from jax.experimental import pallas as pl
from jax.experimental.pallas import tpu as pltpu
from jax.experimental.pallas import tpu_sc as plsc

_N = 10000
_NPAD = 10240
_E = 320000
_EPAD = 327680
_D = 128
_NC, _NS = 2, 16
_CH = 128
_TILE_EDGES = _EPAD // (_NC * _NS)
_NCHUNK = _TILE_EDGES // _CH
_STAGES = 2
_SCHUNK = _NCHUNK // _STAGES
_NBUF = 2
_ITERS = _SCHUNK // _NBUF
_ROWS_PT = _NPAD // _NS

_mesh = plsc.VectorSubcoreMesh(
    core_axis_name="c", subcore_axis_name="s", num_cores=_NC, num_subcores=_NS
)

_DEG_CHUNK = 128
_DEG_NCHUNK = _TILE_EDGES // _DEG_CHUNK


@functools.partial(
    pl.kernel,
    out_type=jax.ShapeDtypeStruct((_NC, _NPAD, 16), jnp.float32),
    mesh=_mesh,
    scratch_types=[
        pltpu.VMEM((_DEG_NCHUNK, _DEG_CHUNK), jnp.int32),
        pltpu.VMEM((_DEG_CHUNK, 16), jnp.float32),
        pltpu.VMEM_SHARED((_NPAD, 16), jnp.float32),
    ],
)
def _deg_kernel(dst_hbm, ones_hbm, zeros_hbm, deg_hbm, dst_v, ones_v, deg_sh):
    c = lax.axis_index("c")
    s = lax.axis_index("s")
    tid = c * _NS + s
    pltpu.sync_copy(dst_hbm.at[tid], dst_v)
    pltpu.sync_copy(ones_hbm, ones_v)
    pltpu.sync_copy(
        zeros_hbm.at[pl.ds(s * _ROWS_PT, _ROWS_PT)],
        deg_sh.at[pl.ds(s * _ROWS_PT, _ROWS_PT)],
    )
    plsc.subcore_barrier()

    def body(j, carry):
        pltpu.sync_copy(ones_v, deg_sh.at[dst_v.at[j]], add=True)
        return carry

    lax.fori_loop(0, _DEG_NCHUNK, body, 0)
    plsc.subcore_barrier()
    pltpu.sync_copy(
        deg_sh.at[pl.ds(s * _ROWS_PT, _ROWS_PT)],
        deg_hbm.at[c, pl.ds(s * _ROWS_PT, _ROWS_PT)],
    )


@functools.partial(
    pl.kernel,
    out_type=jax.ShapeDtypeStruct((_NC, _NPAD, _D), jnp.float32),
    mesh=_mesh,
    scratch_types=[
        pltpu.VMEM((_SCHUNK, _CH), jnp.int32),
        pltpu.VMEM((_SCHUNK, _CH), jnp.int32),
        [pltpu.VMEM((_CH, _D), jnp.float32) for _ in range(_NBUF)],
        pltpu.VMEM_SHARED((_NPAD, _D), jnp.float32),
        [pltpu.SemaphoreType.DMA for _ in range(_NBUF)],
        [pltpu.SemaphoreType.DMA for _ in range(_NBUF)],
    ],
)
def _spmm_kernel(src_hbm, dst_hbm, g_hbm, zeros_hbm, out_hbm,
                 src_v, dst_v, rows, y_sh, gsems, ssems):
    c = lax.axis_index("c")
    s = lax.axis_index("s")
    tid = c * _NS + s
    pltpu.sync_copy(
        zeros_hbm.at[pl.ds(s * _ROWS_PT, _ROWS_PT)],
        y_sh.at[pl.ds(s * _ROWS_PT, _ROWS_PT)],
    )
    plsc.subcore_barrier()

    def body(i, carry):
        gathers = []
        for b in range(_NBUF):
            j = i * _NBUF + b
            gathers.append(
                pltpu.async_copy(g_hbm.at[src_v.at[j]], rows[b], gsems[b])
            )
        scatters = []
        for b in range(_NBUF):
            j = i * _NBUF + b
            gathers[b].wait()
            scatters.append(
                pltpu.async_copy(rows[b], y_sh.at[dst_v.at[j]], ssems[b], add=True)
            )
        for d in scatters:
            d.wait()
        return carry

    for stage in range(_STAGES):
        pltpu.sync_copy(src_hbm.at[tid, pl.ds(stage * _SCHUNK, _SCHUNK)], src_v)
        pltpu.sync_copy(dst_hbm.at[tid, pl.ds(stage * _SCHUNK, _SCHUNK)], dst_v)
        lax.fori_loop(0, _ITERS, body, 0)
    plsc.subcore_barrier()
    pltpu.sync_copy(
        y_sh.at[pl.ds(s * _ROWS_PT, _ROWS_PT)],
        out_hbm.at[c, pl.ds(s * _ROWS_PT, _ROWS_PT)],
    )


_BLK = 256


def _dis_of(deg_ref):
    deg = deg_ref[0, :, 0:1] + deg_ref[1, :, 0:1] + 1.0
    return lax.rsqrt(deg)


def _tc_a_body(x_ref, w_ref, deg_ref, o_ref):
    dis = _dis_of(deg_ref)
    o_ref[...] = jnp.dot(
        x_ref[...], w_ref[...], preferred_element_type=jnp.float32
    ) * dis


_tc_a = pl.pallas_call(
    _tc_a_body,
    grid=(_NPAD // _BLK,),
    in_specs=[
        pl.BlockSpec((_BLK, _D), lambda i: (i, 0)),
        pl.BlockSpec((_D, _D), lambda i: (0, 0)),
        pl.BlockSpec((_NC, _BLK, 16), lambda i: (0, i, 0)),
    ],
    out_specs=pl.BlockSpec((_BLK, _D), lambda i: (i, 0)),
    out_shape=jax.ShapeDtypeStruct((_NPAD, _D), jnp.float32),
)


def _tc_b_body(y_ref, g_ref, deg_ref, b_ref, w_ref, o_ref):
    dis = _dis_of(deg_ref)
    t = (y_ref[0] + y_ref[1] + g_ref[...]) * dis + b_ref[...]
    h = jnp.maximum(t, 0.0)
    o_ref[...] = jnp.dot(h, w_ref[...], preferred_element_type=jnp.float32) * dis


_tc_b = pl.pallas_call(
    _tc_b_body,
    grid=(_NPAD // _BLK,),
    in_specs=[
        pl.BlockSpec((_NC, _BLK, _D), lambda i: (0, i, 0)),
        pl.BlockSpec((_BLK, _D), lambda i: (i, 0)),
        pl.BlockSpec((_NC, _BLK, 16), lambda i: (0, i, 0)),
        pl.BlockSpec((1, _D), lambda i: (0, 0)),
        pl.BlockSpec((_D, _D), lambda i: (0, 0)),
    ],
    out_specs=pl.BlockSpec((_BLK, _D), lambda i: (i, 0)),
    out_shape=jax.ShapeDtypeStruct((_NPAD, _D), jnp.float32),
)


def _tc_c_body(y_ref, g_ref, deg_ref, b_ref, w_ref, bl_ref, o_ref):
    dis = _dis_of(deg_ref)
    t = (y_ref[0] + y_ref[1] + g_ref[...]) * dis + b_ref[...]
    h = jnp.maximum(t, 0.0)
    o_ref[...] = jnp.dot(
        h, w_ref[...], preferred_element_type=jnp.float32
    ) + bl_ref[...]


_tc_c = pl.pallas_call(
    _tc_c_body,
    grid=(_NPAD // _BLK,),
    in_specs=[
        pl.BlockSpec((_NC, _BLK, _D), lambda i: (0, i, 0)),
        pl.BlockSpec((_BLK, _D), lambda i: (i, 0)),
        pl.BlockSpec((_NC, _BLK, 16), lambda i: (0, i, 0)),
        pl.BlockSpec((1, _D), lambda i: (0, 0)),
        pl.BlockSpec((_D, _D), lambda i: (0, 0)),
        pl.BlockSpec((1, _D), lambda i: (0, 0)),
    ],
    out_specs=pl.BlockSpec((_BLK, _D), lambda i: (i, 0)),
    out_shape=jax.ShapeDtypeStruct((_NPAD, _D), jnp.float32),
)


@jax.jit
def _run(x, edge_index, W1, b1, W2, b2, Wl, bl):
    src = edge_index[0].astype(jnp.int32)
    dst = edge_index[1].astype(jnp.int32)
    pad = jnp.full((_EPAD - _E,), _N, jnp.int32)
    src_p = jnp.concatenate([src, pad])
    dst_p = jnp.concatenate([dst, pad])
    src3 = src_p.reshape(_NC * _NS, _NCHUNK, _CH)
    dst3 = dst_p.reshape(_NC * _NS, _NCHUNK, _CH)
    dst3d = dst_p.reshape(_NC * _NS, _DEG_NCHUNK, _DEG_CHUNK)
    x_pad = jnp.concatenate([x, jnp.zeros((_NPAD - _N, _D), x.dtype)])
    ones16 = jnp.ones((_DEG_CHUNK, 16), jnp.float32)
    zeros16 = jnp.zeros((_NPAD, 16), jnp.float32)
    zeros_d = jnp.zeros((_NPAD, _D), jnp.float32)

    deg = _deg_kernel(dst3d, ones16, zeros16)
    g1 = _tc_a(x_pad, W1, deg)
    y1 = _spmm_kernel(src3, dst3, g1, zeros_d)
    g2 = _tc_b(y1, g1, deg, b1.reshape(1, _D), W2)
    y2 = _spmm_kernel(src3, dst3, g2, zeros_d)
    out = _tc_c(y2, g2, deg, b2.reshape(1, _D), Wl, bl.reshape(1, _D))
    return out[:_N]


def kernel(x, edge_index, W1, b1, W2, b2, Wl, bl):
    return _run(x, edge_index, W1, b1, W2, b2, Wl, bl)
